# baseline (speedup 1.0000x reference)
import functools

import jax
import jax.numpy as jnp
from jax import lax
from jax.experimental import pallas as pl
from jax.experimental.pallas import tpu as pltpu
from jax.experimental.pallas import tpu_sc as plsc

LANES = 16
NUM_CORES = 2
NUM_SUBCORES = 16
NUM_WORKERS = NUM_CORES * NUM_SUBCORES


def _make_embed(n_rows: int, seq: int, hidden: int):
    assert n_rows % NUM_WORKERS == 0
    bpw = n_rows // NUM_WORKERS
    assert seq % bpw == 0
    chunks_per_seq = seq // bpw

    mesh = plsc.VectorSubcoreMesh(core_axis_name="c", subcore_axis_name="s")

    @functools.partial(
        pl.kernel,
        mesh=mesh,
        compiler_params=pltpu.CompilerParams(use_tc_tiling_on_sc=False),
        out_type=jax.ShapeDtypeStruct((n_rows, hidden), jnp.float32),
        scratch_types=[
            pltpu.VMEM((bpw,), jnp.int32),
            pltpu.VMEM((bpw, hidden), jnp.float32),
            pltpu.VMEM((bpw, hidden), jnp.float32),
            pltpu.SemaphoreType.DMA,
        ],
    )
    def embed(x_hbm, tok_hbm, pos_hbm, out_hbm, idx_v, rows_v, pos_v, sem):
        wid = lax.axis_index("s") * NUM_CORES + lax.axis_index("c")
        base = wid * bpw
        s_start = (wid % chunks_per_seq) * bpw

        pltpu.sync_copy(x_hbm.at[pl.ds(base, bpw)], idx_v)
        gather = pltpu.async_copy(tok_hbm.at[idx_v], rows_v, sem)
        pltpu.sync_copy(pos_hbm.at[pl.ds(s_start, bpw)], pos_v)
        gather.wait()

        def body(i, _):
            for j in range(hidden // LANES):
                sl = pl.ds(j * LANES, LANES)
                rows_v[i, sl] = rows_v[i, sl] + pos_v[i, sl]
            return ()

        lax.fori_loop(0, bpw, body, ())

        pltpu.sync_copy(rows_v, out_hbm.at[pl.ds(base, bpw)])

    return embed


def kernel(x, token_embedding, position_embedding):
    batch, seq = x.shape
    hidden = token_embedding.shape[1]
    x_flat = x.reshape(-1).astype(jnp.int32)
    fn = _make_embed(batch * seq, seq, hidden)
    out = fn(x_flat, token_embedding, position_embedding)
    return out.reshape(batch, seq, hidden)


# speedup vs baseline: 2.0759x; 2.0759x over previous
import functools

import jax
import jax.numpy as jnp
from jax import lax
from jax.experimental import pallas as pl
from jax.experimental.pallas import tpu as pltpu
from jax.experimental.pallas import tpu_sc as plsc

LANES = 16
NUM_CORES = 2
NUM_SUBCORES = 16
NUM_WORKERS = NUM_CORES * NUM_SUBCORES


def _make_embed(batch: int, seq: int, hidden: int, vocab: int):
    assert hidden % NUM_WORKERS == 0
    fpw = hidden // NUM_WORKERS
    n = batch * seq
    assert seq % LANES == 0

    mesh = plsc.VectorSubcoreMesh(core_axis_name="c", subcore_axis_name="s")

    @functools.partial(
        pl.kernel,
        mesh=mesh,
        compiler_params=pltpu.CompilerParams(needs_layout_passes=False),
        out_type=jax.ShapeDtypeStruct((batch, hidden, seq), jnp.float32),
        scratch_types=[
            pltpu.VMEM((n,), jnp.int32),
            pltpu.VMEM((vocab,), jnp.float32),
            pltpu.VMEM((seq,), jnp.float32),
            pltpu.VMEM((seq,), jnp.float32),
        ],
    )
    def embed(x_hbm, tbl_hbm, pos_hbm, out_hbm, x_v, row_v, pos_v, seg_v):
        wid = lax.axis_index("s") * NUM_CORES + lax.axis_index("c")
        pltpu.sync_copy(x_hbm, x_v)
        for j in range(fpw):
            feat = wid * fpw + j
            pltpu.sync_copy(tbl_hbm.at[feat], row_v)
            pltpu.sync_copy(pos_hbm.at[feat], pos_v)
            for b in range(batch):
                def body(k, _):
                    idxv = x_v[pl.ds(b * seq + k * LANES, LANES)]
                    vals = plsc.load_gather(row_v, [idxv])
                    seg_v[pl.ds(k * LANES, LANES)] = (
                        vals + pos_v[pl.ds(k * LANES, LANES)]
                    )
                    return ()

                lax.fori_loop(0, seq // LANES, body, ())
                pltpu.sync_copy(seg_v, out_hbm.at[b, feat])

    return embed


def kernel(x, token_embedding, position_embedding):
    batch, seq = x.shape
    vocab, hidden = token_embedding.shape
    x_flat = x.reshape(-1).astype(jnp.int32)
    fn = _make_embed(batch, seq, hidden, vocab)
    out = fn(x_flat, token_embedding.T, position_embedding.T)
    return out.transpose(0, 2, 1)


# speedup vs baseline: 2.0812x; 1.0025x over previous
import functools

import jax
import jax.numpy as jnp
from jax import lax
from jax.experimental import pallas as pl
from jax.experimental.pallas import tpu as pltpu
from jax.experimental.pallas import tpu_sc as plsc

LANES = 16
NUM_CORES = 2
NUM_SUBCORES = 16
NUM_WORKERS = NUM_CORES * NUM_SUBCORES


def _make_embed(batch: int, seq: int, hidden: int, vocab: int):
    assert hidden % NUM_WORKERS == 0
    fpw = hidden // NUM_WORKERS
    assert seq % LANES == 0

    mesh = plsc.VectorSubcoreMesh(core_axis_name="c", subcore_axis_name="s")

    @functools.partial(
        pl.kernel,
        mesh=mesh,
        compiler_params=pltpu.CompilerParams(needs_layout_passes=False),
        out_type=jax.ShapeDtypeStruct((batch, hidden, seq), jnp.float32),
        scratch_types=[
            pltpu.VMEM((batch, seq), jnp.int32),
            pltpu.VMEM((vocab,), jnp.float32),
            pltpu.VMEM((seq,), jnp.float32),
            pltpu.VMEM((batch, seq), jnp.float32),
            pltpu.SemaphoreType.DMA,
            pltpu.SemaphoreType.DMA,
            pltpu.SemaphoreType.DMA,
        ],
    )
    def embed(x_hbm, tbl_hbm, pos_hbm, out_hbm, x_v, row_v, pos_v, seg_v,
              row_sem, pos_sem, out_sem):
        wid = lax.axis_index("s") * NUM_CORES + lax.axis_index("c")
        pltpu.sync_copy(x_hbm, x_v)
        for j in range(fpw):
            feat = wid * fpw + j
            row_cp = pltpu.async_copy(tbl_hbm.at[feat], row_v, row_sem)
            pos_cp = pltpu.async_copy(pos_hbm.at[feat], pos_v, pos_sem)
            if j > 0:
                for b in range(batch):
                    out_cps[b].wait()
            pos_cp.wait()
            row_cp.wait()
            out_cps = []
            for b in range(batch):
                def body(k, _):
                    idxv = x_v[b, pl.ds(k * LANES, LANES)]
                    vals = plsc.load_gather(row_v, [idxv])
                    seg_v[b, pl.ds(k * LANES, LANES)] = (
                        vals + pos_v[pl.ds(k * LANES, LANES)]
                    )
                    return ()

                lax.fori_loop(0, seq // LANES, body, (), unroll=4)
                out_cps.append(
                    pltpu.async_copy(seg_v.at[b], out_hbm.at[b, feat], out_sem)
                )
        for b in range(batch):
            out_cps[b].wait()

    return embed


def kernel(x, token_embedding, position_embedding):
    batch, seq = x.shape
    vocab, hidden = token_embedding.shape
    fn = _make_embed(batch, seq, hidden, vocab)
    out = fn(x.astype(jnp.int32), token_embedding.T, position_embedding.T)
    return out.transpose(0, 2, 1)
